# baseline (speedup 1.0000x reference)
import functools

import jax
import jax.numpy as jnp
from jax import lax
from jax.experimental import pallas as pl
from jax.experimental.pallas import tpu as pltpu
from jax.experimental.pallas import tpu_sc as plsc

_N = 10000
_D_IN = 1024
_D_H = 512
_D_ATT = 256
_NC = 10
_NCLS = 4

_B = 128
_NB = 90
_TOT = _NB * _B

_NCORES = 2
_NSUB = 16
_NW = _NCORES * _NSUB
_RPW = _TOT // _NW
_CH = 24
_NBUF = 5


def _sc_gather_body(x_hbm, idx_hbm, out_hbm, idx_v, *rest):
    bufs = rest[:_NBUF]
    gsems = rest[_NBUF:2 * _NBUF]
    ssems = rest[2 * _NBUF:3 * _NBUF]
    wid = lax.axis_index("s") * _NCORES + lax.axis_index("c")
    base = wid * _RPW
    pltpu.sync_copy(idx_hbm.at[pl.ds(base, _RPW)], idx_v)
    nch = _RPW // _CH
    g = [None] * nch
    st = [None] * nch

    def start_gather(j):
        g[j] = pltpu.async_copy(
            x_hbm.at[idx_v.at[pl.ds(j * _CH, _CH)]],
            bufs[j % _NBUF], gsems[j % _NBUF])

    for j in range(min(_NBUF - 1, nch)):
        start_gather(j)
    for i in range(nch):
        g[i].wait()
        st[i] = pltpu.async_copy(
            bufs[i % _NBUF], out_hbm.at[pl.ds(base + i * _CH, _CH)],
            ssems[i % _NBUF])
        j = i + _NBUF - 1
        if j < nch:
            if j - _NBUF >= 0:
                st[j - _NBUF].wait()
            start_gather(j)
    for i in range(nch):
        if st[i] is not None and i >= nch - _NBUF:
            st[i].wait()


def _sc_gather(x_path, gather_idx):
    gathered = pl.kernel(
        _sc_gather_body,
        out_type=jax.ShapeDtypeStruct((_TOT, _D_IN), jnp.float32),
        mesh=plsc.VectorSubcoreMesh(core_axis_name="c", subcore_axis_name="s"),
        scratch_types=(
            [pltpu.VMEM((_RPW,), jnp.int32)]
            + [pltpu.VMEM((_CH, _D_IN), jnp.float32) for _ in range(_NBUF)]
            + [pltpu.SemaphoreType.DMA for _ in range(2 * _NBUF)]
        ),
    )
    return gathered(x_path, gather_idx)


def _expert_body(e_ref, rv_ref, x_ref, w1_ref, b1_ref, w2_ref, b2_ref, out_ref):
    b = pl.program_id(0)
    h1 = jnp.dot(x_ref[...], w1_ref[0], preferred_element_type=jnp.float32)
    h1 = jnp.maximum(h1 + b1_ref[0], 0.0)
    h2 = jnp.dot(h1, w2_ref[0], preferred_element_type=jnp.float32)
    h2 = jnp.maximum(h2 + b2_ref[0], 0.0)
    rv = rv_ref[b]
    mask = (lax.broadcasted_iota(jnp.int32, (_B, 1), 0) < rv).astype(jnp.float32)
    psum = jnp.sum(h2 * mask, axis=0, keepdims=True)[None]
    first = jnp.logical_or(b == 0, e_ref[b] != e_ref[jnp.maximum(b - 1, 0)])

    @pl.when(first)
    def _():
        out_ref[...] = psum

    @pl.when(jnp.logical_not(first))
    def _():
        out_ref[...] = out_ref[...] + psum


def _expert_sums(e, rv, x_sorted, w1, b1, w2, b2):
    grid_spec = pltpu.PrefetchScalarGridSpec(
        num_scalar_prefetch=2,
        grid=(_NB,),
        in_specs=[
            pl.BlockSpec((_B, _D_IN), lambda b, e_ref, rv_ref: (b, 0)),
            pl.BlockSpec((1, _D_IN, _D_H), lambda b, e_ref, rv_ref: (e_ref[b], 0, 0)),
            pl.BlockSpec((1, 1, _D_H), lambda b, e_ref, rv_ref: (e_ref[b], 0, 0)),
            pl.BlockSpec((1, _D_H, _D_H), lambda b, e_ref, rv_ref: (e_ref[b], 0, 0)),
            pl.BlockSpec((1, 1, _D_H), lambda b, e_ref, rv_ref: (e_ref[b], 0, 0)),
        ],
        out_specs=pl.BlockSpec((1, 1, _D_H), lambda b, e_ref, rv_ref: (e_ref[b], 0, 0)),
    )
    out = pl.pallas_call(
        _expert_body,
        grid_spec=grid_spec,
        out_shape=jax.ShapeDtypeStruct((_NC, 1, _D_H), jnp.float32),
    )(e, rv, x_sorted, w1, b1[:, None, :], w2, b2[:, None, :])
    return out[:, 0, :]


def _head_body(hs_ref, cnt_ref, wf_ref, bf_ref, wa_ref, ba_ref, wb_ref, bb_ref,
               wc_ref, bc_ref, wr_ref, br_ref, wcls_ref, bcls_ref,
               haz_ref, s_ref, y_ref):
    cnt = cnt_ref[...]
    hs = hs_ref[...]
    hmean = jnp.where(cnt > 0.0, hs / jnp.maximum(cnt, 1.0), 0.0)
    hp = jnp.dot(hmean, wf_ref[...], preferred_element_type=jnp.float32)
    hp = jnp.maximum(hp + bf_ref[...], 0.0)
    a = jnp.tanh(jnp.dot(hp, wa_ref[...], preferred_element_type=jnp.float32)
                 + ba_ref[...])
    gz = jnp.dot(hp, wb_ref[...], preferred_element_type=jnp.float32) + bb_ref[...]
    g = 1.0 / (1.0 + jnp.exp(-gz))
    att = jnp.dot(a * g, wc_ref[...], preferred_element_type=jnp.float32) \
        + bc_ref[...]
    att = att - jnp.max(att, axis=0, keepdims=True)
    w = jnp.exp(att)
    w = w / jnp.sum(w, axis=0, keepdims=True)
    h_path = jnp.sum(w * hp, axis=0, keepdims=True)
    h_path = jnp.dot(h_path, wr_ref[...], preferred_element_type=jnp.float32) \
        + br_ref[...]
    h_path = jnp.maximum(h_path, 0.0)
    logits = jnp.dot(h_path, wcls_ref[...], preferred_element_type=jnp.float32) \
        + bcls_ref[...]
    haz = 1.0 / (1.0 + jnp.exp(-logits))
    haz_ref[...] = haz
    tri = (lax.broadcasted_iota(jnp.int32, (_NCLS, _NCLS), 0)
           <= lax.broadcasted_iota(jnp.int32, (_NCLS, _NCLS), 1)).astype(jnp.float32)
    s_ref[...] = jnp.exp(
        jnp.dot(jnp.log(1.0 - haz), tri, preferred_element_type=jnp.float32))
    mx = jnp.max(logits, axis=1, keepdims=True)
    ii = lax.broadcasted_iota(jnp.int32, (1, _NCLS), 1)
    y_ref[...] = jnp.min(jnp.where(logits >= mx, ii, _NCLS), axis=1, keepdims=True)


def _head(hs, cnt, wf, bf, wa, ba, wb, bb, wc, bc, wr, br, wcls, bcls):
    return pl.pallas_call(
        _head_body,
        out_shape=(
            jax.ShapeDtypeStruct((1, _NCLS), jnp.float32),
            jax.ShapeDtypeStruct((1, _NCLS), jnp.float32),
            jax.ShapeDtypeStruct((1, 1), jnp.int32),
        ),
    )(hs, cnt, wf, bf, wa, ba, wb, bb, wc, bc, wr, br, wcls, bcls)


def kernel(x_path, cluster_id, phi_W1, phi_b1, phi_W2, phi_b2, Wf, bf, Wa, ba,
           Wb, bb, Wc, bc, Wr, br, Wcls, bcls):
    cid = cluster_id.astype(jnp.int32)
    counts = jnp.sum(
        cid[None, :] == jnp.arange(_NC, dtype=jnp.int32)[:, None],
        axis=1).astype(jnp.int32)
    perm = jnp.argsort(cid).astype(jnp.int32)

    blocks_per = jnp.maximum(-(-counts // _B), 1)
    cumb = jnp.cumsum(blocks_per)
    cumb_before = cumb - blocks_per
    seg_start = jnp.cumsum(counts) - counts
    nblk_total = cumb[_NC - 1]

    bidx = jnp.arange(_NB, dtype=jnp.int32)
    e = jnp.minimum(
        jnp.sum((cumb[None, :] <= bidx[:, None]).astype(jnp.int32), axis=1),
        _NC - 1).astype(jnp.int32)
    eoh = (e[:, None] == jnp.arange(_NC, dtype=jnp.int32)[None, :])
    eohi = eoh.astype(jnp.int32)
    cumb_before_e = jnp.sum(eohi * cumb_before[None, :], axis=1)
    counts_e = jnp.sum(eohi * counts[None, :], axis=1)
    used = bidx < nblk_total
    bk = jnp.where(used, bidx - cumb_before_e, jnp.int32(1 << 20))
    rows_valid = jnp.clip(counts_e - bk * _B, 0, _B).astype(jnp.int32)

    q = jnp.arange(_TOT, dtype=jnp.int32)
    qb = q // _B
    r = q % _B
    eq = jnp.minimum(
        jnp.sum((cumb[None, :] <= qb[:, None]).astype(jnp.int32), axis=1),
        _NC - 1)
    eqoh = (eq[:, None] == jnp.arange(_NC, dtype=jnp.int32)[None, :]).astype(jnp.int32)
    cumb_before_q = jnp.sum(eqoh * cumb_before[None, :], axis=1)
    counts_q = jnp.sum(eqoh * counts[None, :], axis=1)
    seg_q = jnp.sum(eqoh * seg_start[None, :], axis=1)
    bk_q = jnp.where(qb < nblk_total, qb - cumb_before_q, jnp.int32(1 << 20))
    off = bk_q * _B + r
    valid = off < counts_q
    src = jnp.where(valid, seg_q + off, 0)
    gather_idx = jnp.where(valid, perm[src], 0).astype(jnp.int32)

    x_sorted = _sc_gather(x_path, gather_idx)
    h_sum = _expert_sums(e, rows_valid, x_sorted, phi_W1, phi_b1, phi_W2, phi_b2)

    cnt_f = counts.astype(jnp.float32)[:, None]
    haz, s, y = _head(
        h_sum, cnt_f, Wf, bf[None, :], Wa, ba[None, :], Wb, bb[None, :],
        Wc, bc[None, :], Wr, br[None, :], Wcls, bcls[None, :])
    return (haz, s, y)


# speedup vs baseline: 1.7210x; 1.7210x over previous
import functools

import jax
import jax.numpy as jnp
from jax import lax
from jax.experimental import pallas as pl
from jax.experimental.pallas import tpu as pltpu
from jax.experimental.pallas import tpu_sc as plsc

_N = 10000
_D_IN = 1024
_D_H = 512
_D_ATT = 256
_NC = 10
_NCLS = 4

_B = 128
_NB = 90
_TOT = _NB * _B

_NCORES = 2
_NSUB = 16
_NW = _NCORES * _NSUB
_RPW = _TOT // _NW
_CH = 24
_NBUF = 5


def _sc_gather_body(x_hbm, idx_hbm, out_hbm, idx_v, *rest):
    bufs = rest[:_NBUF]
    gsems = rest[_NBUF:2 * _NBUF]
    ssems = rest[2 * _NBUF:3 * _NBUF]
    wid = lax.axis_index("s") * _NCORES + lax.axis_index("c")
    base = wid * _RPW
    pltpu.sync_copy(idx_hbm.at[pl.ds(base, _RPW)], idx_v)
    nch = _RPW // _CH
    g = [None] * nch
    st = [None] * nch

    def start_gather(j):
        g[j] = pltpu.async_copy(
            x_hbm.at[idx_v.at[pl.ds(j * _CH, _CH)]],
            bufs[j % _NBUF], gsems[j % _NBUF])

    for j in range(min(_NBUF - 1, nch)):
        start_gather(j)
    for i in range(nch):
        g[i].wait()
        st[i] = pltpu.async_copy(
            bufs[i % _NBUF], out_hbm.at[pl.ds(base + i * _CH, _CH)],
            ssems[i % _NBUF])
        j = i + _NBUF - 1
        if j < nch:
            if j - _NBUF >= 0:
                st[j - _NBUF].wait()
            start_gather(j)
    for i in range(nch):
        if st[i] is not None and i >= nch - _NBUF:
            st[i].wait()


def _sc_gather(x_path, gather_idx):
    gathered = pl.kernel(
        _sc_gather_body,
        out_type=jax.ShapeDtypeStruct((_TOT, _D_IN), jnp.float32),
        mesh=plsc.VectorSubcoreMesh(core_axis_name="c", subcore_axis_name="s"),
        scratch_types=(
            [pltpu.VMEM((_RPW,), jnp.int32)]
            + [pltpu.VMEM((_CH, _D_IN), jnp.float32) for _ in range(_NBUF)]
            + [pltpu.SemaphoreType.DMA for _ in range(2 * _NBUF)]
        ),
    )
    return gathered(x_path, gather_idx)


def _expert_body(e_ref, rv_ref, x_ref, w1_ref, b1_ref, w2_ref, b2_ref, out_ref):
    b = pl.program_id(0)
    h1 = jnp.dot(x_ref[...], w1_ref[0], preferred_element_type=jnp.float32)
    h1 = jnp.maximum(h1 + b1_ref[0], 0.0)
    h2 = jnp.dot(h1, w2_ref[0], preferred_element_type=jnp.float32)
    h2 = jnp.maximum(h2 + b2_ref[0], 0.0)
    rv = rv_ref[b]
    mask = (lax.broadcasted_iota(jnp.int32, (_B, 1), 0) < rv).astype(jnp.float32)
    psum = jnp.sum(h2 * mask, axis=0, keepdims=True)[None]
    first = jnp.logical_or(b == 0, e_ref[b] != e_ref[jnp.maximum(b - 1, 0)])

    @pl.when(first)
    def _():
        out_ref[...] = psum

    @pl.when(jnp.logical_not(first))
    def _():
        out_ref[...] = out_ref[...] + psum


def _expert_sums(e, rv, x_sorted, w1, b1, w2, b2):
    grid_spec = pltpu.PrefetchScalarGridSpec(
        num_scalar_prefetch=2,
        grid=(_NB,),
        in_specs=[
            pl.BlockSpec((_B, _D_IN), lambda b, e_ref, rv_ref: (b, 0)),
            pl.BlockSpec((1, _D_IN, _D_H), lambda b, e_ref, rv_ref: (e_ref[b], 0, 0)),
            pl.BlockSpec((1, 1, _D_H), lambda b, e_ref, rv_ref: (e_ref[b], 0, 0)),
            pl.BlockSpec((1, _D_H, _D_H), lambda b, e_ref, rv_ref: (e_ref[b], 0, 0)),
            pl.BlockSpec((1, 1, _D_H), lambda b, e_ref, rv_ref: (e_ref[b], 0, 0)),
        ],
        out_specs=pl.BlockSpec((1, 1, _D_H), lambda b, e_ref, rv_ref: (e_ref[b], 0, 0)),
    )
    out = pl.pallas_call(
        _expert_body,
        grid_spec=grid_spec,
        out_shape=jax.ShapeDtypeStruct((_NC, 1, _D_H), jnp.float32),
    )(e, rv, x_sorted, w1, b1[:, None, :], w2, b2[:, None, :])
    return out[:, 0, :]


def _head_body(hs_ref, cnt_ref, wf_ref, bf_ref, wa_ref, ba_ref, wb_ref, bb_ref,
               wc_ref, bc_ref, wr_ref, br_ref, wcls_ref, bcls_ref,
               haz_ref, s_ref, y_ref):
    cnt = cnt_ref[...]
    hs = hs_ref[...]
    hmean = jnp.where(cnt > 0.0, hs / jnp.maximum(cnt, 1.0), 0.0)
    hp = jnp.dot(hmean, wf_ref[...], preferred_element_type=jnp.float32)
    hp = jnp.maximum(hp + bf_ref[...], 0.0)
    a = jnp.tanh(jnp.dot(hp, wa_ref[...], preferred_element_type=jnp.float32)
                 + ba_ref[...])
    gz = jnp.dot(hp, wb_ref[...], preferred_element_type=jnp.float32) + bb_ref[...]
    g = 1.0 / (1.0 + jnp.exp(-gz))
    att = jnp.dot(a * g, wc_ref[...], preferred_element_type=jnp.float32) \
        + bc_ref[...]
    att = att - jnp.max(att, axis=0, keepdims=True)
    w = jnp.exp(att)
    w = w / jnp.sum(w, axis=0, keepdims=True)
    h_path = jnp.sum(w * hp, axis=0, keepdims=True)
    h_path = jnp.dot(h_path, wr_ref[...], preferred_element_type=jnp.float32) \
        + br_ref[...]
    h_path = jnp.maximum(h_path, 0.0)
    logits = jnp.dot(h_path, wcls_ref[...], preferred_element_type=jnp.float32) \
        + bcls_ref[...]
    haz = 1.0 / (1.0 + jnp.exp(-logits))
    haz_ref[...] = haz
    tri = (lax.broadcasted_iota(jnp.int32, (_NCLS, _NCLS), 0)
           <= lax.broadcasted_iota(jnp.int32, (_NCLS, _NCLS), 1)).astype(jnp.float32)
    s_ref[...] = jnp.exp(
        jnp.dot(jnp.log(1.0 - haz), tri, preferred_element_type=jnp.float32))
    mx = jnp.max(logits, axis=1, keepdims=True)
    ii = lax.broadcasted_iota(jnp.int32, (1, _NCLS), 1)
    y_ref[...] = jnp.min(jnp.where(logits >= mx, ii, _NCLS), axis=1, keepdims=True)


def _head(hs, cnt, wf, bf, wa, ba, wb, bb, wc, bc, wr, br, wcls, bcls):
    return pl.pallas_call(
        _head_body,
        out_shape=(
            jax.ShapeDtypeStruct((1, _NCLS), jnp.float32),
            jax.ShapeDtypeStruct((1, _NCLS), jnp.float32),
            jax.ShapeDtypeStruct((1, 1), jnp.int32),
        ),
    )(hs, cnt, wf, bf, wa, ba, wb, bb, wc, bc, wr, br, wcls, bcls)


def kernel(x_path, cluster_id, phi_W1, phi_b1, phi_W2, phi_b2, Wf, bf, Wa, ba,
           Wb, bb, Wc, bc, Wr, br, Wcls, bcls):
    cid = cluster_id.astype(jnp.int32)
    counts = jnp.sum(
        cid[None, :] == jnp.arange(_NC, dtype=jnp.int32)[:, None],
        axis=1).astype(jnp.int32)
    perm = jnp.argsort(cid).astype(jnp.int32)

    blocks_per = jnp.maximum(-(-counts // _B), 1)
    cumb = jnp.cumsum(blocks_per)
    cumb_before = cumb - blocks_per
    seg_start = jnp.cumsum(counts) - counts
    nblk_total = cumb[_NC - 1]

    bidx = jnp.arange(_NB, dtype=jnp.int32)
    e = jnp.minimum(
        jnp.sum((cumb[None, :] <= bidx[:, None]).astype(jnp.int32), axis=1),
        _NC - 1).astype(jnp.int32)
    eoh = (e[:, None] == jnp.arange(_NC, dtype=jnp.int32)[None, :])
    eohi = eoh.astype(jnp.int32)
    cumb_before_e = jnp.sum(eohi * cumb_before[None, :], axis=1)
    counts_e = jnp.sum(eohi * counts[None, :], axis=1)
    used = bidx < nblk_total
    bk = jnp.where(used, bidx - cumb_before_e, jnp.int32(1 << 20))
    rows_valid = jnp.clip(counts_e - bk * _B, 0, _B).astype(jnp.int32)

    q = jnp.arange(_TOT, dtype=jnp.int32)
    qb = q // _B
    r = q % _B
    eq = jnp.minimum(
        jnp.sum((cumb[None, :] <= qb[:, None]).astype(jnp.int32), axis=1),
        _NC - 1)
    eqoh = (eq[:, None] == jnp.arange(_NC, dtype=jnp.int32)[None, :]).astype(jnp.int32)
    cumb_before_q = jnp.sum(eqoh * cumb_before[None, :], axis=1)
    counts_q = jnp.sum(eqoh * counts[None, :], axis=1)
    seg_q = jnp.sum(eqoh * seg_start[None, :], axis=1)
    bk_q = jnp.where(qb < nblk_total, qb - cumb_before_q, jnp.int32(1 << 20))
    off = bk_q * _B + r
    valid = off < counts_q
    src = jnp.where(valid, seg_q + off, 0)
    gather_idx = jnp.where(valid, perm[src], q & 8191).astype(jnp.int32)

    x_sorted = _sc_gather(x_path, gather_idx)
    h_sum = _expert_sums(e, rows_valid, x_sorted, phi_W1, phi_b1, phi_W2, phi_b2)

    cnt_f = counts.astype(jnp.float32)[:, None]
    haz, s, y = _head(
        h_sum, cnt_f, Wf, bf[None, :], Wa, ba[None, :], Wb, bb[None, :],
        Wc, bc[None, :], Wr, br[None, :], Wcls, bcls[None, :])
    return (haz, s, y)


# speedup vs baseline: 1.8722x; 1.0879x over previous
import functools

import jax
import jax.numpy as jnp
from jax import lax
from jax.experimental import pallas as pl
from jax.experimental.pallas import tpu as pltpu
from jax.experimental.pallas import tpu_sc as plsc

_N = 10000
_D_IN = 1024
_D_H = 512
_D_ATT = 256
_NC = 10
_NCLS = 4

_B = 256
_NB = 50
_TOT = _NB * _B

_NCORES = 2
_NSUB = 16
_NW = _NCORES * _NSUB
_RPW = _TOT // _NW
_CH = 40
_NBUF = 3


def _sc_gather_body(x_hbm, idx_hbm, out_hbm, idx_v, *rest):
    bufs = rest[:_NBUF]
    gsems = rest[_NBUF:2 * _NBUF]
    ssems = rest[2 * _NBUF:3 * _NBUF]
    wid = lax.axis_index("s") * _NCORES + lax.axis_index("c")
    base = wid * _RPW
    pltpu.sync_copy(idx_hbm.at[pl.ds(base, _RPW)], idx_v)
    nch = _RPW // _CH
    g = [None] * nch
    st = [None] * nch

    def start_gather(j):
        g[j] = pltpu.async_copy(
            x_hbm.at[idx_v.at[pl.ds(j * _CH, _CH)]],
            bufs[j % _NBUF], gsems[j % _NBUF])

    for j in range(min(_NBUF - 1, nch)):
        start_gather(j)
    for i in range(nch):
        g[i].wait()
        st[i] = pltpu.async_copy(
            bufs[i % _NBUF], out_hbm.at[pl.ds(base + i * _CH, _CH)],
            ssems[i % _NBUF])
        j = i + _NBUF - 1
        if j < nch:
            if j - _NBUF >= 0:
                st[j - _NBUF].wait()
            start_gather(j)
    for i in range(nch):
        if st[i] is not None and i >= nch - _NBUF:
            st[i].wait()


def _sc_gather(x_path, gather_idx):
    gathered = pl.kernel(
        _sc_gather_body,
        out_type=jax.ShapeDtypeStruct((_TOT, _D_IN), jnp.float32),
        mesh=plsc.VectorSubcoreMesh(core_axis_name="c", subcore_axis_name="s"),
        scratch_types=(
            [pltpu.VMEM((_RPW,), jnp.int32)]
            + [pltpu.VMEM((_CH, _D_IN), jnp.float32) for _ in range(_NBUF)]
            + [pltpu.SemaphoreType.DMA for _ in range(2 * _NBUF)]
        ),
    )
    return gathered(x_path, gather_idx)


def _expert_body(e_ref, rv_ref, x_ref, w1_ref, b1_ref, w2_ref, b2_ref, out_ref):
    b = pl.program_id(0)
    h1 = jnp.dot(x_ref[...], w1_ref[0], preferred_element_type=jnp.float32)
    h1 = jnp.maximum(h1 + b1_ref[0], 0.0)
    h2 = jnp.dot(h1, w2_ref[0], preferred_element_type=jnp.float32)
    h2 = jnp.maximum(h2 + b2_ref[0], 0.0)
    rv = rv_ref[b]
    mask = (lax.broadcasted_iota(jnp.int32, (_B, 1), 0) < rv).astype(jnp.float32)
    psum = jnp.sum(h2 * mask, axis=0, keepdims=True)[None]
    first = jnp.logical_or(b == 0, e_ref[b] != e_ref[jnp.maximum(b - 1, 0)])

    @pl.when(first)
    def _():
        out_ref[...] = psum

    @pl.when(jnp.logical_not(first))
    def _():
        out_ref[...] = out_ref[...] + psum


def _expert_sums(e, rv, x_sorted, w1, b1, w2, b2):
    grid_spec = pltpu.PrefetchScalarGridSpec(
        num_scalar_prefetch=2,
        grid=(_NB,),
        in_specs=[
            pl.BlockSpec((_B, _D_IN), lambda b, e_ref, rv_ref: (b, 0)),
            pl.BlockSpec((1, _D_IN, _D_H), lambda b, e_ref, rv_ref: (e_ref[b], 0, 0)),
            pl.BlockSpec((1, 1, _D_H), lambda b, e_ref, rv_ref: (e_ref[b], 0, 0)),
            pl.BlockSpec((1, _D_H, _D_H), lambda b, e_ref, rv_ref: (e_ref[b], 0, 0)),
            pl.BlockSpec((1, 1, _D_H), lambda b, e_ref, rv_ref: (e_ref[b], 0, 0)),
        ],
        out_specs=pl.BlockSpec((1, 1, _D_H), lambda b, e_ref, rv_ref: (e_ref[b], 0, 0)),
    )
    out = pl.pallas_call(
        _expert_body,
        grid_spec=grid_spec,
        out_shape=jax.ShapeDtypeStruct((_NC, 1, _D_H), jnp.float32),
    )(e, rv, x_sorted, w1, b1[:, None, :], w2, b2[:, None, :])
    return out[:, 0, :]


def _head_body(hs_ref, cnt_ref, wf_ref, bf_ref, wa_ref, ba_ref, wb_ref, bb_ref,
               wc_ref, bc_ref, wr_ref, br_ref, wcls_ref, bcls_ref,
               haz_ref, s_ref, y_ref):
    cnt = cnt_ref[...]
    hs = hs_ref[...]
    hmean = jnp.where(cnt > 0.0, hs / jnp.maximum(cnt, 1.0), 0.0)
    hp = jnp.dot(hmean, wf_ref[...], preferred_element_type=jnp.float32)
    hp = jnp.maximum(hp + bf_ref[...], 0.0)
    a = jnp.tanh(jnp.dot(hp, wa_ref[...], preferred_element_type=jnp.float32)
                 + ba_ref[...])
    gz = jnp.dot(hp, wb_ref[...], preferred_element_type=jnp.float32) + bb_ref[...]
    g = 1.0 / (1.0 + jnp.exp(-gz))
    att = jnp.dot(a * g, wc_ref[...], preferred_element_type=jnp.float32) \
        + bc_ref[...]
    att = att - jnp.max(att, axis=0, keepdims=True)
    w = jnp.exp(att)
    w = w / jnp.sum(w, axis=0, keepdims=True)
    h_path = jnp.sum(w * hp, axis=0, keepdims=True)
    h_path = jnp.dot(h_path, wr_ref[...], preferred_element_type=jnp.float32) \
        + br_ref[...]
    h_path = jnp.maximum(h_path, 0.0)
    logits = jnp.dot(h_path, wcls_ref[...], preferred_element_type=jnp.float32) \
        + bcls_ref[...]
    haz = 1.0 / (1.0 + jnp.exp(-logits))
    haz_ref[...] = haz
    tri = (lax.broadcasted_iota(jnp.int32, (_NCLS, _NCLS), 0)
           <= lax.broadcasted_iota(jnp.int32, (_NCLS, _NCLS), 1)).astype(jnp.float32)
    s_ref[...] = jnp.exp(
        jnp.dot(jnp.log(1.0 - haz), tri, preferred_element_type=jnp.float32))
    mx = jnp.max(logits, axis=1, keepdims=True)
    ii = lax.broadcasted_iota(jnp.int32, (1, _NCLS), 1)
    y_ref[...] = jnp.min(jnp.where(logits >= mx, ii, _NCLS), axis=1, keepdims=True)


def _head(hs, cnt, wf, bf, wa, ba, wb, bb, wc, bc, wr, br, wcls, bcls):
    return pl.pallas_call(
        _head_body,
        out_shape=(
            jax.ShapeDtypeStruct((1, _NCLS), jnp.float32),
            jax.ShapeDtypeStruct((1, _NCLS), jnp.float32),
            jax.ShapeDtypeStruct((1, 1), jnp.int32),
        ),
    )(hs, cnt, wf, bf, wa, ba, wb, bb, wc, bc, wr, br, wcls, bcls)


def kernel(x_path, cluster_id, phi_W1, phi_b1, phi_W2, phi_b2, Wf, bf, Wa, ba,
           Wb, bb, Wc, bc, Wr, br, Wcls, bcls):
    cid = cluster_id.astype(jnp.int32)
    counts = jnp.sum(
        cid[None, :] == jnp.arange(_NC, dtype=jnp.int32)[:, None],
        axis=1).astype(jnp.int32)
    perm = jnp.argsort(cid).astype(jnp.int32)

    blocks_per = jnp.maximum(-(-counts // _B), 1)
    cumb = jnp.cumsum(blocks_per)
    cumb_before = cumb - blocks_per
    seg_start = jnp.cumsum(counts) - counts
    nblk_total = cumb[_NC - 1]

    bidx = jnp.arange(_NB, dtype=jnp.int32)
    e = jnp.minimum(
        jnp.sum((cumb[None, :] <= bidx[:, None]).astype(jnp.int32), axis=1),
        _NC - 1).astype(jnp.int32)
    eoh = (e[:, None] == jnp.arange(_NC, dtype=jnp.int32)[None, :])
    eohi = eoh.astype(jnp.int32)
    cumb_before_e = jnp.sum(eohi * cumb_before[None, :], axis=1)
    counts_e = jnp.sum(eohi * counts[None, :], axis=1)
    used = bidx < nblk_total
    bk = jnp.where(used, bidx - cumb_before_e, jnp.int32(1 << 20))
    rows_valid = jnp.clip(counts_e - bk * _B, 0, _B).astype(jnp.int32)

    q = jnp.arange(_TOT, dtype=jnp.int32)
    qb = q // _B
    r = q % _B
    eq = jnp.minimum(
        jnp.sum((cumb[None, :] <= qb[:, None]).astype(jnp.int32), axis=1),
        _NC - 1)
    eqoh = (eq[:, None] == jnp.arange(_NC, dtype=jnp.int32)[None, :]).astype(jnp.int32)
    cumb_before_q = jnp.sum(eqoh * cumb_before[None, :], axis=1)
    counts_q = jnp.sum(eqoh * counts[None, :], axis=1)
    seg_q = jnp.sum(eqoh * seg_start[None, :], axis=1)
    bk_q = jnp.where(qb < nblk_total, qb - cumb_before_q, jnp.int32(1 << 20))
    off = bk_q * _B + r
    valid = off < counts_q
    src = jnp.where(valid, seg_q + off, 0)
    gather_idx = jnp.where(valid, perm[src], q & 8191).astype(jnp.int32)

    x_sorted = _sc_gather(x_path, gather_idx)
    h_sum = _expert_sums(e, rows_valid, x_sorted, phi_W1, phi_b1, phi_W2, phi_b2)

    cnt_f = counts.astype(jnp.float32)[:, None]
    haz, s, y = _head(
        h_sum, cnt_f, Wf, bf[None, :], Wa, ba[None, :], Wb, bb[None, :],
        Wc, bc[None, :], Wr, br[None, :], Wcls, bcls[None, :])
    return (haz, s, y)


# speedup vs baseline: 2.3819x; 1.2723x over previous
import functools

import jax
import jax.numpy as jnp
from jax import lax
from jax.experimental import pallas as pl
from jax.experimental.pallas import tpu as pltpu
from jax.experimental.pallas import tpu_sc as plsc

_N = 10000
_D_IN = 1024
_D_H = 512
_D_ATT = 256
_NC = 10
_NCLS = 4

_B = 256
_NBLK = 40
_NB = _NBLK + _NC
_TOT = _NBLK * _B

_NCORES = 2
_NSUB = 16
_NW = _NCORES * _NSUB
_RPW = _TOT // _NW
_CH = 40
_NBUF = 3


def _sc_gather_body(x_hbm, idx_hbm, out_hbm, idx_v, *rest):
    bufs = rest[:_NBUF]
    gsems = rest[_NBUF:2 * _NBUF]
    ssems = rest[2 * _NBUF:3 * _NBUF]
    wid = lax.axis_index("s") * _NCORES + lax.axis_index("c")
    base = wid * _RPW
    pltpu.sync_copy(idx_hbm.at[pl.ds(base, _RPW)], idx_v)
    nch = _RPW // _CH
    g = [None] * nch
    st = [None] * nch

    def start_gather(j):
        g[j] = pltpu.async_copy(
            x_hbm.at[idx_v.at[pl.ds(j * _CH, _CH)]],
            bufs[j % _NBUF], gsems[j % _NBUF])

    for j in range(min(_NBUF - 1, nch)):
        start_gather(j)
    for i in range(nch):
        g[i].wait()
        st[i] = pltpu.async_copy(
            bufs[i % _NBUF], out_hbm.at[pl.ds(base + i * _CH, _CH)],
            ssems[i % _NBUF])
        j = i + _NBUF - 1
        if j < nch:
            if j - _NBUF >= 0:
                st[j - _NBUF].wait()
            start_gather(j)
    for i in range(nch):
        if st[i] is not None and i >= nch - _NBUF:
            st[i].wait()


def _sc_gather(x_path, gather_idx):
    gathered = pl.kernel(
        _sc_gather_body,
        out_type=jax.ShapeDtypeStruct((_TOT, _D_IN), jnp.float32),
        mesh=plsc.VectorSubcoreMesh(core_axis_name="c", subcore_axis_name="s"),
        scratch_types=(
            [pltpu.VMEM((_RPW,), jnp.int32)]
            + [pltpu.VMEM((_CH, _D_IN), jnp.float32) for _ in range(_NBUF)]
            + [pltpu.SemaphoreType.DMA for _ in range(2 * _NBUF)]
        ),
    )
    return gathered(x_path, gather_idx)


def _expert_body(e_ref, qb_ref, lo_ref, hi_ref, x_ref, w1_ref, b1_ref, w2_ref,
                 b2_ref, out_ref):
    b = pl.program_id(0)
    h1 = jnp.dot(x_ref[...], w1_ref[0], preferred_element_type=jnp.float32)
    h1 = jnp.maximum(h1 + b1_ref[0], 0.0)
    h2 = jnp.dot(h1, w2_ref[0], preferred_element_type=jnp.float32)
    h2 = jnp.maximum(h2 + b2_ref[0], 0.0)
    ii = lax.broadcasted_iota(jnp.int32, (_B, 1), 0)
    mask = ((ii >= lo_ref[b]) & (ii < hi_ref[b])).astype(jnp.float32)
    psum = jnp.sum(h2 * mask, axis=0, keepdims=True)[None]
    first = jnp.logical_or(b == 0, e_ref[b] != e_ref[jnp.maximum(b - 1, 0)])

    @pl.when(first)
    def _():
        out_ref[...] = psum

    @pl.when(jnp.logical_not(first))
    def _():
        out_ref[...] = out_ref[...] + psum


def _expert_sums(e, qb, lo, hi, x_sorted, w1, b1, w2, b2):
    grid_spec = pltpu.PrefetchScalarGridSpec(
        num_scalar_prefetch=4,
        grid=(_NB,),
        in_specs=[
            pl.BlockSpec((_B, _D_IN), lambda b, e_r, qb_r, lo_r, hi_r: (qb_r[b], 0)),
            pl.BlockSpec((1, _D_IN, _D_H), lambda b, e_r, qb_r, lo_r, hi_r: (e_r[b], 0, 0)),
            pl.BlockSpec((1, 1, _D_H), lambda b, e_r, qb_r, lo_r, hi_r: (e_r[b], 0, 0)),
            pl.BlockSpec((1, _D_H, _D_H), lambda b, e_r, qb_r, lo_r, hi_r: (e_r[b], 0, 0)),
            pl.BlockSpec((1, 1, _D_H), lambda b, e_r, qb_r, lo_r, hi_r: (e_r[b], 0, 0)),
        ],
        out_specs=pl.BlockSpec((1, 1, _D_H), lambda b, e_r, qb_r, lo_r, hi_r: (e_r[b], 0, 0)),
    )
    out = pl.pallas_call(
        _expert_body,
        grid_spec=grid_spec,
        out_shape=jax.ShapeDtypeStruct((_NC, 1, _D_H), jnp.float32),
    )(e, qb, lo, hi, x_sorted, w1, b1[:, None, :], w2, b2[:, None, :])
    return out[:, 0, :]


def _head_body(hs_ref, cnt_ref, wf_ref, bf_ref, wa_ref, ba_ref, wb_ref, bb_ref,
               wc_ref, bc_ref, wr_ref, br_ref, wcls_ref, bcls_ref,
               haz_ref, s_ref, y_ref):
    cnt = cnt_ref[...]
    hs = hs_ref[...]
    hmean = jnp.where(cnt > 0.0, hs / jnp.maximum(cnt, 1.0), 0.0)
    hp = jnp.dot(hmean, wf_ref[...], preferred_element_type=jnp.float32)
    hp = jnp.maximum(hp + bf_ref[...], 0.0)
    a = jnp.tanh(jnp.dot(hp, wa_ref[...], preferred_element_type=jnp.float32)
                 + ba_ref[...])
    gz = jnp.dot(hp, wb_ref[...], preferred_element_type=jnp.float32) + bb_ref[...]
    g = 1.0 / (1.0 + jnp.exp(-gz))
    att = jnp.dot(a * g, wc_ref[...], preferred_element_type=jnp.float32) \
        + bc_ref[...]
    att = att - jnp.max(att, axis=0, keepdims=True)
    w = jnp.exp(att)
    w = w / jnp.sum(w, axis=0, keepdims=True)
    h_path = jnp.sum(w * hp, axis=0, keepdims=True)
    h_path = jnp.dot(h_path, wr_ref[...], preferred_element_type=jnp.float32) \
        + br_ref[...]
    h_path = jnp.maximum(h_path, 0.0)
    logits = jnp.dot(h_path, wcls_ref[...], preferred_element_type=jnp.float32) \
        + bcls_ref[...]
    haz = 1.0 / (1.0 + jnp.exp(-logits))
    haz_ref[...] = haz
    tri = (lax.broadcasted_iota(jnp.int32, (_NCLS, _NCLS), 0)
           <= lax.broadcasted_iota(jnp.int32, (_NCLS, _NCLS), 1)).astype(jnp.float32)
    s_ref[...] = jnp.exp(
        jnp.dot(jnp.log(1.0 - haz), tri, preferred_element_type=jnp.float32))
    mx = jnp.max(logits, axis=1, keepdims=True)
    ii = lax.broadcasted_iota(jnp.int32, (1, _NCLS), 1)
    y_ref[...] = jnp.min(jnp.where(logits >= mx, ii, _NCLS), axis=1, keepdims=True)


def _head(hs, cnt, wf, bf, wa, ba, wb, bb, wc, bc, wr, br, wcls, bcls):
    return pl.pallas_call(
        _head_body,
        out_shape=(
            jax.ShapeDtypeStruct((1, _NCLS), jnp.float32),
            jax.ShapeDtypeStruct((1, _NCLS), jnp.float32),
            jax.ShapeDtypeStruct((1, 1), jnp.int32),
        ),
    )(hs, cnt, wf, bf, wa, ba, wb, bb, wc, bc, wr, br, wcls, bcls)


def kernel(x_path, cluster_id, phi_W1, phi_b1, phi_W2, phi_b2, Wf, bf, Wa, ba,
           Wb, bb, Wc, bc, Wr, br, Wcls, bcls):
    cid = cluster_id.astype(jnp.int32)
    counts = jnp.sum(
        cid[None, :] == jnp.arange(_NC, dtype=jnp.int32)[:, None],
        axis=1).astype(jnp.int32)
    perm = jnp.argsort(cid).astype(jnp.int32)

    seg_start = jnp.cumsum(counts) - counts
    seg_end = seg_start + counts
    sb = seg_start // _B
    ebk = jnp.maximum(-(-seg_end // _B), sb + 1)
    pairs_per = ebk - sb
    cump = jnp.cumsum(pairs_per)
    cump_before = cump - pairs_per
    npairs = cump[_NC - 1]

    sidx = jnp.arange(_NB, dtype=jnp.int32)
    e = jnp.minimum(
        jnp.sum((cump[None, :] <= sidx[:, None]).astype(jnp.int32), axis=1),
        _NC - 1).astype(jnp.int32)
    eohi = (e[:, None] == jnp.arange(_NC, dtype=jnp.int32)[None, :]).astype(jnp.int32)
    cump_before_e = jnp.sum(eohi * cump_before[None, :], axis=1)
    sb_e = jnp.sum(eohi * sb[None, :], axis=1)
    st_e = jnp.sum(eohi * seg_start[None, :], axis=1)
    en_e = jnp.sum(eohi * seg_end[None, :], axis=1)
    used = sidx < npairs
    qb = jnp.where(used, sb_e + (sidx - cump_before_e), _NBLK - 1).astype(jnp.int32)
    lo = jnp.where(used, jnp.clip(st_e - qb * _B, 0, _B), 0).astype(jnp.int32)
    hi = jnp.where(used, jnp.clip(en_e - qb * _B, 0, _B), 0).astype(jnp.int32)

    pad = (jnp.arange(_TOT - _N, dtype=jnp.int32) * 7) & 8191
    gather_idx = jnp.concatenate([perm, pad])

    x_sorted = _sc_gather(x_path, gather_idx)
    h_sum = _expert_sums(e, qb, lo, hi, x_sorted, phi_W1, phi_b1, phi_W2, phi_b2)

    cnt_f = counts.astype(jnp.float32)[:, None]
    haz, s, y = _head(
        h_sum, cnt_f, Wf, bf[None, :], Wa, ba[None, :], Wb, bb[None, :],
        Wc, bc[None, :], Wr, br[None, :], Wcls, bcls[None, :])
    return (haz, s, y)


# speedup vs baseline: 2.6911x; 1.1298x over previous
import functools

import jax
import jax.numpy as jnp
from jax import lax
from jax.experimental import pallas as pl
from jax.experimental.pallas import tpu as pltpu
from jax.experimental.pallas import tpu_sc as plsc

_N = 10000
_D_IN = 1024
_D_H = 512
_D_ATT = 256
_NC = 10
_NCLS = 4

_B = 512
_NBLK = 20
_NB = _NBLK + _NC
_TOT = _NBLK * _B

_NCORES = 2
_NSUB = 16
_NW = _NCORES * _NSUB
_RPW = _TOT // _NW
_CH = 40
_NBUF = 3


def _sc_gather_body(x_hbm, idx_hbm, out_hbm, idx_v, *rest):
    bufs = rest[:_NBUF]
    gsems = rest[_NBUF:2 * _NBUF]
    ssems = rest[2 * _NBUF:3 * _NBUF]
    wid = lax.axis_index("s") * _NCORES + lax.axis_index("c")
    base = wid * _RPW
    pltpu.sync_copy(idx_hbm.at[pl.ds(base, _RPW)], idx_v)
    nch = _RPW // _CH
    g = [None] * nch
    st = [None] * nch

    def start_gather(j):
        g[j] = pltpu.async_copy(
            x_hbm.at[idx_v.at[pl.ds(j * _CH, _CH)]],
            bufs[j % _NBUF], gsems[j % _NBUF])

    for j in range(min(_NBUF - 1, nch)):
        start_gather(j)
    for i in range(nch):
        g[i].wait()
        st[i] = pltpu.async_copy(
            bufs[i % _NBUF], out_hbm.at[pl.ds(base + i * _CH, _CH)],
            ssems[i % _NBUF])
        j = i + _NBUF - 1
        if j < nch:
            if j - _NBUF >= 0:
                st[j - _NBUF].wait()
            start_gather(j)
    for i in range(nch):
        if st[i] is not None and i >= nch - _NBUF:
            st[i].wait()


def _sc_gather(x_path, gather_idx):
    gathered = pl.kernel(
        _sc_gather_body,
        out_type=jax.ShapeDtypeStruct((_TOT, _D_IN), jnp.float32),
        mesh=plsc.VectorSubcoreMesh(core_axis_name="c", subcore_axis_name="s"),
        scratch_types=(
            [pltpu.VMEM((_RPW,), jnp.int32)]
            + [pltpu.VMEM((_CH, _D_IN), jnp.float32) for _ in range(_NBUF)]
            + [pltpu.SemaphoreType.DMA for _ in range(2 * _NBUF)]
        ),
    )
    return gathered(x_path, gather_idx)


def _expert_body(e_ref, qb_ref, lo_ref, hi_ref, x_ref, w1_ref, b1_ref, w2_ref,
                 b2_ref, out_ref):
    b = pl.program_id(0)
    h1 = jnp.dot(x_ref[...], w1_ref[0], preferred_element_type=jnp.float32)
    h1 = jnp.maximum(h1 + b1_ref[0], 0.0)
    h2 = jnp.dot(h1, w2_ref[0], preferred_element_type=jnp.float32)
    h2 = jnp.maximum(h2 + b2_ref[0], 0.0)
    ii = lax.broadcasted_iota(jnp.int32, (_B, 1), 0)
    mask = ((ii >= lo_ref[b]) & (ii < hi_ref[b])).astype(jnp.float32)
    psum = jnp.sum(h2 * mask, axis=0, keepdims=True)[None]
    first = jnp.logical_or(b == 0, e_ref[b] != e_ref[jnp.maximum(b - 1, 0)])

    @pl.when(first)
    def _():
        out_ref[...] = psum

    @pl.when(jnp.logical_not(first))
    def _():
        out_ref[...] = out_ref[...] + psum


def _expert_sums(e, qb, lo, hi, x_sorted, w1, b1, w2, b2):
    grid_spec = pltpu.PrefetchScalarGridSpec(
        num_scalar_prefetch=4,
        grid=(_NB,),
        in_specs=[
            pl.BlockSpec((_B, _D_IN), lambda b, e_r, qb_r, lo_r, hi_r: (qb_r[b], 0)),
            pl.BlockSpec((1, _D_IN, _D_H), lambda b, e_r, qb_r, lo_r, hi_r: (e_r[b], 0, 0)),
            pl.BlockSpec((1, 1, _D_H), lambda b, e_r, qb_r, lo_r, hi_r: (e_r[b], 0, 0)),
            pl.BlockSpec((1, _D_H, _D_H), lambda b, e_r, qb_r, lo_r, hi_r: (e_r[b], 0, 0)),
            pl.BlockSpec((1, 1, _D_H), lambda b, e_r, qb_r, lo_r, hi_r: (e_r[b], 0, 0)),
        ],
        out_specs=pl.BlockSpec((1, 1, _D_H), lambda b, e_r, qb_r, lo_r, hi_r: (e_r[b], 0, 0)),
    )
    out = pl.pallas_call(
        _expert_body,
        grid_spec=grid_spec,
        out_shape=jax.ShapeDtypeStruct((_NC, 1, _D_H), jnp.float32),
    )(e, qb, lo, hi, x_sorted, w1, b1[:, None, :], w2, b2[:, None, :])
    return out[:, 0, :]


def _head_body(hs_ref, cnt_ref, wf_ref, bf_ref, wa_ref, ba_ref, wb_ref, bb_ref,
               wc_ref, bc_ref, wr_ref, br_ref, wcls_ref, bcls_ref,
               haz_ref, s_ref, y_ref):
    cnt = cnt_ref[...]
    hs = hs_ref[...]
    hmean = jnp.where(cnt > 0.0, hs / jnp.maximum(cnt, 1.0), 0.0)
    hp = jnp.dot(hmean, wf_ref[...], preferred_element_type=jnp.float32)
    hp = jnp.maximum(hp + bf_ref[...], 0.0)
    a = jnp.tanh(jnp.dot(hp, wa_ref[...], preferred_element_type=jnp.float32)
                 + ba_ref[...])
    gz = jnp.dot(hp, wb_ref[...], preferred_element_type=jnp.float32) + bb_ref[...]
    g = 1.0 / (1.0 + jnp.exp(-gz))
    att = jnp.dot(a * g, wc_ref[...], preferred_element_type=jnp.float32) \
        + bc_ref[...]
    att = att - jnp.max(att, axis=0, keepdims=True)
    w = jnp.exp(att)
    w = w / jnp.sum(w, axis=0, keepdims=True)
    h_path = jnp.sum(w * hp, axis=0, keepdims=True)
    h_path = jnp.dot(h_path, wr_ref[...], preferred_element_type=jnp.float32) \
        + br_ref[...]
    h_path = jnp.maximum(h_path, 0.0)
    logits = jnp.dot(h_path, wcls_ref[...], preferred_element_type=jnp.float32) \
        + bcls_ref[...]
    haz = 1.0 / (1.0 + jnp.exp(-logits))
    haz_ref[...] = haz
    tri = (lax.broadcasted_iota(jnp.int32, (_NCLS, _NCLS), 0)
           <= lax.broadcasted_iota(jnp.int32, (_NCLS, _NCLS), 1)).astype(jnp.float32)
    s_ref[...] = jnp.exp(
        jnp.dot(jnp.log(1.0 - haz), tri, preferred_element_type=jnp.float32))
    mx = jnp.max(logits, axis=1, keepdims=True)
    ii = lax.broadcasted_iota(jnp.int32, (1, _NCLS), 1)
    y_ref[...] = jnp.min(jnp.where(logits >= mx, ii, _NCLS), axis=1, keepdims=True)


def _head(hs, cnt, wf, bf, wa, ba, wb, bb, wc, bc, wr, br, wcls, bcls):
    return pl.pallas_call(
        _head_body,
        out_shape=(
            jax.ShapeDtypeStruct((1, _NCLS), jnp.float32),
            jax.ShapeDtypeStruct((1, _NCLS), jnp.float32),
            jax.ShapeDtypeStruct((1, 1), jnp.int32),
        ),
    )(hs, cnt, wf, bf, wa, ba, wb, bb, wc, bc, wr, br, wcls, bcls)


def kernel(x_path, cluster_id, phi_W1, phi_b1, phi_W2, phi_b2, Wf, bf, Wa, ba,
           Wb, bb, Wc, bc, Wr, br, Wcls, bcls):
    cid = cluster_id.astype(jnp.int32)
    counts = jnp.sum(
        cid[None, :] == jnp.arange(_NC, dtype=jnp.int32)[:, None],
        axis=1).astype(jnp.int32)
    perm = jnp.argsort(cid).astype(jnp.int32)

    seg_start = jnp.cumsum(counts) - counts
    seg_end = seg_start + counts
    sb = seg_start // _B
    ebk = jnp.maximum(-(-seg_end // _B), sb + 1)
    pairs_per = ebk - sb
    cump = jnp.cumsum(pairs_per)
    cump_before = cump - pairs_per
    npairs = cump[_NC - 1]

    sidx = jnp.arange(_NB, dtype=jnp.int32)
    e = jnp.minimum(
        jnp.sum((cump[None, :] <= sidx[:, None]).astype(jnp.int32), axis=1),
        _NC - 1).astype(jnp.int32)
    eohi = (e[:, None] == jnp.arange(_NC, dtype=jnp.int32)[None, :]).astype(jnp.int32)
    cump_before_e = jnp.sum(eohi * cump_before[None, :], axis=1)
    sb_e = jnp.sum(eohi * sb[None, :], axis=1)
    st_e = jnp.sum(eohi * seg_start[None, :], axis=1)
    en_e = jnp.sum(eohi * seg_end[None, :], axis=1)
    used = sidx < npairs
    qb = jnp.where(used, sb_e + (sidx - cump_before_e), _NBLK - 1).astype(jnp.int32)
    lo = jnp.where(used, jnp.clip(st_e - qb * _B, 0, _B), 0).astype(jnp.int32)
    hi = jnp.where(used, jnp.clip(en_e - qb * _B, 0, _B), 0).astype(jnp.int32)

    pad = (jnp.arange(_TOT - _N, dtype=jnp.int32) * 7) & 8191
    gather_idx = jnp.concatenate([perm, pad])

    x_sorted = _sc_gather(x_path, gather_idx)
    h_sum = _expert_sums(e, qb, lo, hi, x_sorted, phi_W1, phi_b1, phi_W2, phi_b2)

    cnt_f = counts.astype(jnp.float32)[:, None]
    haz, s, y = _head(
        h_sum, cnt_f, Wf, bf[None, :], Wa, ba[None, :], Wb, bb[None, :],
        Wc, bc[None, :], Wr, br[None, :], Wcls, bcls[None, :])
    return (haz, s, y)
